# baseline (speedup 1.0000x reference)
import functools

import jax
import jax.numpy as jnp
from jax.experimental import pallas as pl
from jax.experimental.pallas import tpu as pltpu

_GP = 4
_CH = 4


def _body(x_ref, out_ref, stash_ref, b2_ref, stage_ref, mn_ref, mx_ref,
          in_sems, b2_sems, out_sems, *, nb, h, wcc):
    i = pl.program_id(0)
    ng = nb // _GP
    base = i * _CH * wcc

    def in_copy(c, g):
        return pltpu.make_async_copy(
            x_ref.at[pl.ds(g * _GP, _GP), 0, 0, :, pl.ds(base + c * wcc, wcc)],
            stash_ref.at[c, pl.ds(g * _GP, _GP)], in_sems.at[c, g])

    def b2_copy(c):
        return pltpu.make_async_copy(
            x_ref.at[0, 0, 2, :, pl.ds(base + c * wcc, wcc)], b2_ref.at[c],
            b2_sems.at[c])

    def out_copy(k):
        c_k = jax.lax.div(k, ng)
        g_k = jax.lax.rem(k, ng)
        return pltpu.make_async_copy(
            stage_ref.at[jax.lax.rem(k, 2)],
            out_ref.at[pl.ds(g_k * _GP, _GP), 0, :,
                       pl.ds(base + c_k * wcc, wcc)],
            out_sems.at[jax.lax.rem(k, 2)])

    for c in range(_CH):
        b2_copy(c).start()

        def _issue(g, _, c=c):
            in_copy(c, g).start()
            return ()

        jax.lax.fori_loop(0, ng, _issue, (), unroll=True)

    for c in range(_CH):
        in_copy(c, 0).wait()
        b2_copy(c).wait()
        s0 = b2_ref[c] - stash_ref[c, 0]
        stash_ref[c, 0] = s0
        rest = stash_ref[c, pl.ds(1, _GP - 1)].reshape((_GP - 1) * h, wcc)
        mn_ref[...] = jnp.minimum(
            jnp.min(s0, axis=0, keepdims=True),
            jnp.min(rest, axis=0, keepdims=True))
        mx_ref[...] = jnp.maximum(
            jnp.max(s0, axis=0, keepdims=True),
            jnp.max(rest, axis=0, keepdims=True))

        def _reduce(g, _, c=c):
            in_copy(c, g).wait()
            s = stash_ref[c, pl.ds(g * _GP, _GP)].reshape(_GP * h, wcc)
            mn_ref[...] = jnp.minimum(
                mn_ref[...], jnp.min(s, axis=0, keepdims=True))
            mx_ref[...] = jnp.maximum(
                mx_ref[...], jnp.max(s, axis=0, keepdims=True))
            return ()

        jax.lax.fori_loop(1, ng, _reduce, ())

        mn = mn_ref[...]
        rng = mx_ref[...] - mn
        inv = 1.0 / jnp.where(rng == 0.0, 1.0, rng)

        def _store(g, _, c=c):
            k = c * ng + g

            @pl.when(k >= 2)
            def _():
                out_copy(k - 2).wait()

            s = stash_ref[c, pl.ds(g * _GP, _GP)].reshape(_GP * h, wcc)
            stage_ref[jax.lax.rem(k, 2)] = ((s - mn) * inv).reshape(_GP, h, wcc)
            out_copy(k).start(priority=1)
            return ()

        jax.lax.fori_loop(0, ng, _store, ())

    out_copy(_CH * ng - 2).wait()
    out_copy(_CH * ng - 1).wait()


def kernel(x):
    bs, c, n, h, w = x.shape
    nb = bs * c
    cores = 2
    wcc = w // (cores * _CH)

    body = functools.partial(_body, nb=nb, h=h, wcc=wcc)
    out = pl.pallas_call(
        body,
        grid=(cores,),
        in_specs=[pl.BlockSpec(memory_space=pl.ANY)],
        out_specs=pl.BlockSpec(memory_space=pl.ANY),
        out_shape=jax.ShapeDtypeStruct((nb, 1, h, w), x.dtype),
        scratch_shapes=[
            pltpu.VMEM((_CH, nb, h, wcc), jnp.float32),
            pltpu.VMEM((_CH, h, wcc), jnp.float32),
            pltpu.VMEM((2, _GP, h, wcc), jnp.float32),
            pltpu.VMEM((1, wcc), jnp.float32),
            pltpu.VMEM((1, wcc), jnp.float32),
            pltpu.SemaphoreType.DMA((_CH, nb // _GP)),
            pltpu.SemaphoreType.DMA((_CH,)),
            pltpu.SemaphoreType.DMA((2,)),
        ],
        compiler_params=pltpu.CompilerParams(
            dimension_semantics=("parallel",),
            vmem_limit_bytes=56 * 1024 * 1024,
        ),
    )(x)
    return out.reshape(bs, c, h, w)


# speedup vs baseline: 1.0858x; 1.0858x over previous
import functools

import jax
import jax.numpy as jnp
from jax.experimental import pallas as pl
from jax.experimental.pallas import tpu as pltpu

_RG = 8
_GP = 4
_CH = 2


def _body(x_ref, out_ref, stash_ref, b2_ref, stage_ref, mn_ref, mx_ref,
          in_sems, b2_sems, out_sems, *, nb, h, wcc):
    i = pl.program_id(0)
    nrg = nb // _RG
    ng = nb // _GP
    base = i * _CH * wcc

    def in_copy(c, g):
        return pltpu.make_async_copy(
            x_ref.at[pl.ds(g * _RG, _RG), 0, 0, :, pl.ds(base + c * wcc, wcc)],
            stash_ref.at[c, pl.ds(g * _RG, _RG)], in_sems.at[c, g])

    def b2_copy(c):
        return pltpu.make_async_copy(
            x_ref.at[0, 0, 2, :, pl.ds(base + c * wcc, wcc)], b2_ref.at[c],
            b2_sems.at[c])

    def out_copy(k):
        c_k = jax.lax.div(k, ng)
        g_k = jax.lax.rem(k, ng)
        return pltpu.make_async_copy(
            stage_ref.at[jax.lax.rem(k, 2)],
            out_ref.at[pl.ds(g_k * _GP, _GP), 0, :,
                       pl.ds(base + c_k * wcc, wcc)],
            out_sems.at[jax.lax.rem(k, 2)])

    for c in range(_CH):
        b2_copy(c).start()

        def _issue(g, _, c=c):
            in_copy(c, g).start()
            return ()

        jax.lax.fori_loop(0, nrg, _issue, (), unroll=True)

    for c in range(_CH):
        in_copy(c, 0).wait()
        b2_copy(c).wait()
        s0 = b2_ref[c] - stash_ref[c, 0]
        stash_ref[c, 0] = s0
        rest = stash_ref[c, pl.ds(1, _RG - 1)].reshape((_RG - 1) * h, wcc)
        mn_ref[...] = jnp.minimum(
            jnp.min(s0, axis=0, keepdims=True),
            jnp.min(rest, axis=0, keepdims=True))
        mx_ref[...] = jnp.maximum(
            jnp.max(s0, axis=0, keepdims=True),
            jnp.max(rest, axis=0, keepdims=True))

        def _reduce(g, _, c=c):
            in_copy(c, g).wait()
            s = stash_ref[c, pl.ds(g * _RG, _RG)].reshape(_RG * h, wcc)
            mn_ref[...] = jnp.minimum(
                mn_ref[...], jnp.min(s, axis=0, keepdims=True))
            mx_ref[...] = jnp.maximum(
                mx_ref[...], jnp.max(s, axis=0, keepdims=True))
            return ()

        jax.lax.fori_loop(1, nrg, _reduce, ())

        mn = mn_ref[...]
        rng = mx_ref[...] - mn
        inv = 1.0 / jnp.where(rng == 0.0, 1.0, rng)

        def _store(g, _, c=c):
            k = c * ng + g

            @pl.when(k >= 2)
            def _():
                out_copy(k - 2).wait()

            s = stash_ref[c, pl.ds(g * _GP, _GP)].reshape(_GP * h, wcc)
            stage_ref[jax.lax.rem(k, 2)] = ((s - mn) * inv).reshape(_GP, h, wcc)
            out_copy(k).start(priority=1)
            return ()

        jax.lax.fori_loop(0, ng, _store, ())

    out_copy(_CH * ng - 2).wait()
    out_copy(_CH * ng - 1).wait()


def kernel(x):
    bs, c, n, h, w = x.shape
    nb = bs * c
    cores = 2
    wcc = w // (cores * _CH)

    body = functools.partial(_body, nb=nb, h=h, wcc=wcc)
    out = pl.pallas_call(
        body,
        grid=(cores,),
        in_specs=[pl.BlockSpec(memory_space=pl.ANY)],
        out_specs=pl.BlockSpec(memory_space=pl.ANY),
        out_shape=jax.ShapeDtypeStruct((nb, 1, h, w), x.dtype),
        scratch_shapes=[
            pltpu.VMEM((_CH, nb, h, wcc), jnp.float32),
            pltpu.VMEM((_CH, h, wcc), jnp.float32),
            pltpu.VMEM((2, _GP, h, wcc), jnp.float32),
            pltpu.VMEM((1, wcc), jnp.float32),
            pltpu.VMEM((1, wcc), jnp.float32),
            pltpu.SemaphoreType.DMA((_CH, nb // _RG)),
            pltpu.SemaphoreType.DMA((_CH,)),
            pltpu.SemaphoreType.DMA((2,)),
        ],
        compiler_params=pltpu.CompilerParams(
            dimension_semantics=("parallel",),
            vmem_limit_bytes=56 * 1024 * 1024,
        ),
    )(x)
    return out.reshape(bs, c, h, w)


# speedup vs baseline: 1.1042x; 1.0169x over previous
import functools

import jax
import jax.numpy as jnp
from jax.experimental import pallas as pl
from jax.experimental.pallas import tpu as pltpu

_RG = 8
_GP = 4
_CH = 2


def _body(x_ref, out_ref, stash_ref, b2_ref, stage_ref, mn_ref, mx_ref,
          in_sems, b2_sems, out_sems, *, nb, h, wcc):
    i = pl.program_id(0)
    nrg = nb // _RG
    ng = nb // _GP
    base = i * _CH * wcc

    def in_copy(c, g):
        return pltpu.make_async_copy(
            x_ref.at[pl.ds(g * _RG, _RG), 0, 0, :, pl.ds(base + c * wcc, wcc)],
            stash_ref.at[c, pl.ds(g * _RG, _RG)], in_sems.at[c, g])

    def b2_copy(c):
        return pltpu.make_async_copy(
            x_ref.at[0, 0, 2, :, pl.ds(base + c * wcc, wcc)], b2_ref.at[c],
            b2_sems.at[c])

    def out_copy(k):
        c_k = jax.lax.div(k, ng)
        g_k = jax.lax.rem(k, ng)
        return pltpu.make_async_copy(
            stage_ref.at[jax.lax.rem(k, 4)],
            out_ref.at[pl.ds(g_k * _GP, _GP), 0, :,
                       pl.ds(base + c_k * wcc, wcc)],
            out_sems.at[jax.lax.rem(k, 4)])

    for c in range(_CH):
        b2_copy(c).start()

        def _issue(g, _, c=c):
            in_copy(c, g).start()
            return ()

        jax.lax.fori_loop(0, nrg, _issue, (), unroll=True)

    for c in range(_CH):
        in_copy(c, 0).wait()
        b2_copy(c).wait()
        s0 = b2_ref[c] - stash_ref[c, 0]
        stash_ref[c, 0] = s0
        rest = stash_ref[c, pl.ds(1, _RG - 1)].reshape((_RG - 1) * h, wcc)
        mn_ref[...] = jnp.minimum(
            jnp.min(s0, axis=0, keepdims=True),
            jnp.min(rest, axis=0, keepdims=True))
        mx_ref[...] = jnp.maximum(
            jnp.max(s0, axis=0, keepdims=True),
            jnp.max(rest, axis=0, keepdims=True))

        def _reduce(g, _, c=c):
            in_copy(c, g).wait()
            s = stash_ref[c, pl.ds(g * _RG, _RG)].reshape(_RG * h, wcc)
            mn_ref[...] = jnp.minimum(
                mn_ref[...], jnp.min(s, axis=0, keepdims=True))
            mx_ref[...] = jnp.maximum(
                mx_ref[...], jnp.max(s, axis=0, keepdims=True))
            return ()

        jax.lax.fori_loop(1, nrg, _reduce, ())

        mn = mn_ref[...]
        rng = mx_ref[...] - mn
        inv = 1.0 / jnp.where(rng == 0.0, 1.0, rng)

        def _store(g, _, c=c):
            k = c * ng + g

            @pl.when(k >= 4)
            def _():
                out_copy(k - 4).wait()

            s = stash_ref[c, pl.ds(g * _GP, _GP)].reshape(_GP * h, wcc)
            stage_ref[jax.lax.rem(k, 4)] = ((s - mn) * inv).reshape(_GP, h, wcc)
            out_copy(k).start(priority=1)
            return ()

        jax.lax.fori_loop(0, ng, _store, ())

    for k in range(_CH * ng - 4, _CH * ng):
        out_copy(k).wait()


def kernel(x):
    bs, c, n, h, w = x.shape
    nb = bs * c
    cores = 2
    wcc = w // (cores * _CH)

    body = functools.partial(_body, nb=nb, h=h, wcc=wcc)
    out = pl.pallas_call(
        body,
        grid=(cores,),
        in_specs=[pl.BlockSpec(memory_space=pl.ANY)],
        out_specs=pl.BlockSpec(memory_space=pl.ANY),
        out_shape=jax.ShapeDtypeStruct((nb, 1, h, w), x.dtype),
        scratch_shapes=[
            pltpu.VMEM((_CH, nb, h, wcc), jnp.float32),
            pltpu.VMEM((_CH, h, wcc), jnp.float32),
            pltpu.VMEM((4, _GP, h, wcc), jnp.float32),
            pltpu.VMEM((1, wcc), jnp.float32),
            pltpu.VMEM((1, wcc), jnp.float32),
            pltpu.SemaphoreType.DMA((_CH, nb // _RG)),
            pltpu.SemaphoreType.DMA((_CH,)),
            pltpu.SemaphoreType.DMA((4,)),
        ],
        compiler_params=pltpu.CompilerParams(
            dimension_semantics=("parallel",),
            vmem_limit_bytes=56 * 1024 * 1024,
        ),
    )(x)
    return out.reshape(bs, c, h, w)
